# baseline (speedup 1.0000x reference)
import functools

import jax
import jax.numpy as jnp
from jax import lax
from jax.experimental import pallas as pl
from jax.experimental.pallas import tpu as pltpu
from jax.experimental.pallas import tpu_sc as plsc

_V_PAD = 1024


def _sigmoid_t_body(x_ref, o_ref):
    o_ref[...] = jax.nn.sigmoid(x_ref[...]).T


def _sigmoid_table_t(logits):
    n_rows, d = logits.shape
    return pl.pallas_call(
        _sigmoid_t_body,
        out_shape=jax.ShapeDtypeStruct((d, n_rows), logits.dtype),
    )(logits)


@functools.cache
def _make_gather(n_rows, d, b):
    nc, ns = 2, 16
    nw = nc * ns
    gc = 8
    n_groups = d // gc
    max_gpw = -(-n_groups // nw)
    bch = 2048
    n_bch = b // bch
    mesh = plsc.VectorSubcoreMesh(core_axis_name="c", subcore_axis_name="s")

    @functools.partial(
        pl.kernel,
        mesh=mesh,
        out_type=jax.ShapeDtypeStruct((d, b), jnp.float32),
        scratch_types=[
            pltpu.VMEM((b,), jnp.int32),
            pltpu.VMEM((gc, n_rows), jnp.float32),
            pltpu.VMEM((gc, bch), jnp.float32),
            pltpu.VMEM((gc, bch), jnp.float32),
            pltpu.SemaphoreType.DMA,
            pltpu.SemaphoreType.DMA,
        ],
        compiler_params=pltpu.CompilerParams(
            use_tc_tiling_on_sc=True, needs_layout_passes=False),
    )
    def gather_kernel(table_hbm, idx_hbm, out_hbm, idx_v, tbl, ob0, ob1,
                      s0, s1):
        wid = lax.axis_index("s") * nc + lax.axis_index("c")
        pltpu.sync_copy(idx_hbm, idx_v)
        obufs = (ob0, ob1)
        sems = (s0, s1)
        cvecs = [jnp.full((16,), cl, dtype=jnp.int32) for cl in range(gc)]

        for gi in range(max_gpw):
            g = wid + nw * gi

            @pl.when(g < n_groups)
            def _process_group():
                c0 = pl.multiple_of(g * gc, gc)
                pltpu.sync_copy(table_hbm.at[pl.ds(c0, gc)], tbl)
                scatters = [None, None]
                for bc in range(n_bch):
                    bsel = bc % 2
                    obuf = obufs[bsel]
                    if scatters[bsel] is not None:
                        scatters[bsel].wait()

                    def bgrp_body(k, carry, bc=bc, obuf=obuf):
                        idxv = idx_v[pl.ds(bc * bch + k * 16, 16)]
                        for cl in range(gc):
                            v = plsc.load_gather(tbl, [cvecs[cl], idxv])
                            obuf[cl, pl.ds(k * 16, 16)] = v
                        return carry

                    lax.fori_loop(0, bch // 16, bgrp_body, 0)
                    scatters[bsel] = pltpu.async_copy(
                        obuf,
                        out_hbm.at[pl.ds(c0, gc), pl.ds(bc * bch, bch)],
                        sems[bsel],
                    )
                scatters[0].wait()
                scatters[1].wait()

    return gather_kernel


def kernel(indices, logits):
    n_rows, d = logits.shape
    (b,) = indices.shape
    table_t = _sigmoid_table_t(logits)
    out_t = _make_gather(n_rows, d, b)(table_t, indices)
    return out_t.T


# speedup vs baseline: 1.5637x; 1.5637x over previous
import functools

import jax
import jax.numpy as jnp
from jax import lax
from jax.experimental import pallas as pl
from jax.experimental.pallas import tpu as pltpu
from jax.experimental.pallas import tpu_sc as plsc

_V_PAD = 1024


def _sigmoid_t_body(x_ref, o_ref):
    o_ref[...] = jax.nn.sigmoid(x_ref[...]).T


def _sigmoid_table_t(logits):
    n_rows, d = logits.shape
    return pl.pallas_call(
        _sigmoid_t_body,
        out_shape=jax.ShapeDtypeStruct((d, n_rows), logits.dtype),
    )(logits)


@functools.cache
def _make_gather(n_rows, d, b):
    nc, ns = 2, 16
    nw = nc * ns
    gc = 8
    n_groups = d // gc
    max_gpw = -(-n_groups // nw)
    bch = 2048
    n_bch = b // bch
    mesh = plsc.VectorSubcoreMesh(core_axis_name="c", subcore_axis_name="s")

    @functools.partial(
        pl.kernel,
        mesh=mesh,
        out_type=jax.ShapeDtypeStruct((d, b), jnp.float32),
        scratch_types=[
            pltpu.VMEM((b,), jnp.int32),
            pltpu.VMEM((gc * n_rows,), jnp.float32),
            pltpu.VMEM((gc, bch), jnp.float32),
            pltpu.VMEM((gc, bch), jnp.float32),
            pltpu.SemaphoreType.DMA,
            pltpu.SemaphoreType.DMA,
        ],
        compiler_params=pltpu.CompilerParams(
            use_tc_tiling_on_sc=True, needs_layout_passes=False),
    )
    def gather_kernel(table_hbm, idx_hbm, out_hbm, idx_v, tbl, ob0, ob1,
                      s0, s1):
        wid = lax.axis_index("s") * nc + lax.axis_index("c")
        pltpu.sync_copy(idx_hbm, idx_v)
        obufs = (ob0, ob1)
        sems = (s0, s1)

        for gi in range(max_gpw):
            g = wid + nw * gi

            @pl.when(g < n_groups)
            def _process_group():
                c0 = pl.multiple_of(g * gc, gc)
                for cl in range(gc):
                    pltpu.sync_copy(table_hbm.at[c0 + cl],
                                    tbl.at[pl.ds(cl * n_rows, n_rows)])
                scatters = [None, None]
                for bc in range(n_bch):
                    bsel = bc % 2
                    obuf = obufs[bsel]
                    if scatters[bsel] is not None:
                        scatters[bsel].wait()

                    def bgrp_body(k, carry, bc=bc, obuf=obuf):
                        idxv = idx_v[pl.ds(bc * bch + k * 16, 16)]
                        vals = [
                            plsc.load_gather(
                                tbl.at[pl.ds(cl * n_rows, n_rows)], [idxv])
                            for cl in range(gc)
                        ]
                        for cl in range(gc):
                            obuf[cl, pl.ds(k * 16, 16)] = vals[cl]
                        return carry

                    lax.fori_loop(0, bch // 16, bgrp_body, 0)
                    scatters[bsel] = pltpu.async_copy(
                        obuf,
                        out_hbm.at[pl.ds(c0, gc), pl.ds(bc * bch, bch)],
                        sems[bsel],
                    )
                scatters[0].wait()
                scatters[1].wait()

    return gather_kernel


def kernel(indices, logits):
    n_rows, d = logits.shape
    (b,) = indices.shape
    table_t = _sigmoid_table_t(logits)
    out_t = _make_gather(n_rows, d, b)(table_t, indices)
    return out_t.T


# speedup vs baseline: 1.8340x; 1.1728x over previous
import functools

import jax
import jax.numpy as jnp
from jax import lax
from jax.experimental import pallas as pl
from jax.experimental.pallas import tpu as pltpu
from jax.experimental.pallas import tpu_sc as plsc


def _sigmoid_body(x_ref, o_ref):
    o_ref[...] = jax.nn.sigmoid(x_ref[...])


def _sigmoid_table_t(logits_t):
    return pl.pallas_call(
        _sigmoid_body,
        out_shape=jax.ShapeDtypeStruct(logits_t.shape, logits_t.dtype),
    )(logits_t)


@functools.cache
def _make_gather(n_rows, d, b):
    nc, ns = 2, 16
    nw = nc * ns
    gc = 8
    n_groups = d // gc
    max_gpw = -(-n_groups // nw)
    bch = 2048
    n_bch = b // bch
    mesh = plsc.VectorSubcoreMesh(core_axis_name="c", subcore_axis_name="s")

    @functools.partial(
        pl.kernel,
        mesh=mesh,
        out_type=jax.ShapeDtypeStruct((d, b), jnp.float32),
        scratch_types=[
            pltpu.VMEM((b,), jnp.int32),
            pltpu.VMEM((gc * n_rows,), jnp.float32),
            pltpu.VMEM((gc, bch), jnp.float32),
            pltpu.VMEM((gc, bch), jnp.float32),
            pltpu.SemaphoreType.DMA,
            pltpu.SemaphoreType.DMA,
        ],
        compiler_params=pltpu.CompilerParams(
            use_tc_tiling_on_sc=True, needs_layout_passes=False),
    )
    def gather_kernel(table_hbm, idx_hbm, out_hbm, idx_v, tbl, ob0, ob1,
                      s0, s1):
        wid = lax.axis_index("s") * nc + lax.axis_index("c")
        pltpu.sync_copy(idx_hbm, idx_v)
        obufs = (ob0, ob1)
        sems = (s0, s1)

        for gi in range(max_gpw):
            g = wid + nw * gi

            @pl.when(g < n_groups)
            def _process_group(g=g):
                c0 = pl.multiple_of(g * gc, gc)
                for cl in range(gc):
                    pltpu.sync_copy(table_hbm.at[c0 + cl],
                                    tbl.at[pl.ds(cl * n_rows, n_rows)])
                scatters = [None, None]
                for bc in range(n_bch):
                    bsel = bc % 2
                    obuf = obufs[bsel]
                    if scatters[bsel] is not None:
                        scatters[bsel].wait()

                    def bgrp_body(k, carry, bc=bc, obuf=obuf):
                        b0 = bc * bch + k * 32
                        idxv0 = idx_v[pl.ds(b0, 16)]
                        idxv1 = idx_v[pl.ds(b0 + 16, 16)]
                        slices = [tbl.at[pl.ds(cl * n_rows, n_rows)]
                                  for cl in range(gc)]
                        vals0 = [plsc.load_gather(slices[cl], [idxv0])
                                 for cl in range(gc)]
                        vals1 = [plsc.load_gather(slices[cl], [idxv1])
                                 for cl in range(gc)]
                        for cl in range(gc):
                            obuf[cl, pl.ds(k * 32, 16)] = vals0[cl]
                        for cl in range(gc):
                            obuf[cl, pl.ds(k * 32 + 16, 16)] = vals1[cl]
                        return carry

                    lax.fori_loop(0, bch // 32, bgrp_body, 0)
                    scatters[bsel] = pltpu.async_copy(
                        obuf,
                        out_hbm.at[pl.ds(c0, gc), pl.ds(bc * bch, bch)],
                        sems[bsel],
                    )
                scatters[0].wait()
                scatters[1].wait()

    return gather_kernel


def kernel(indices, logits):
    n_rows, d = logits.shape
    (b,) = indices.shape
    table_t = _sigmoid_table_t(logits.T)
    out_t = _make_gather(n_rows, d, b)(table_t, indices)
    return out_t.T
